# baseline (speedup 1.0000x reference)
import functools

import jax
import jax.numpy as jnp
from jax import lax
from jax.experimental import pallas as pl
from jax.experimental.pallas import tpu as pltpu
from jax.experimental.pallas import tpu_sc as plsc

_B = 16384
_N_NUM = 13
_N_CAT = 26
_VOCAB = 100000 + 1
_DIM = 32
_PDIM = 128
_FLAT = _B * _N_CAT

_NC = 2
_NS = 16
_NW = _NC * _NS
_ROWS_PER_W = _FLAT // _NW
_CB = 832
_N_CHUNKS = _ROWS_PER_W // _CB


def _body(idx_hbm, tab_hbm, out_hbm, idx_v, rows_v, sem):
    cid = lax.axis_index("c")
    sid = lax.axis_index("s")
    wid = sid * _NC + cid
    w_base = wid * _ROWS_PER_W

    def chunk(g, carry):
        base = w_base + g * _CB
        pltpu.sync_copy(idx_hbm.at[pl.ds(base, _CB)], idx_v)
        pltpu.async_copy(tab_hbm.at[idx_v], rows_v, sem).wait()
        pltpu.sync_copy(rows_v, out_hbm.at[pl.ds(base, _CB)])
        return carry

    lax.fori_loop(0, _N_CHUNKS, chunk, 0)


@jax.jit
def _run(idx_flat, tab_pad):
    mesh = plsc.VectorSubcoreMesh(core_axis_name="c", subcore_axis_name="s")
    k = functools.partial(
        pl.kernel,
        mesh=mesh,
        out_type=jax.ShapeDtypeStruct((_FLAT, _PDIM), jnp.float32),
        scratch_types=[
            pltpu.VMEM((_CB,), jnp.int32),
            pltpu.VMEM((_CB, _PDIM), jnp.float32),
            pltpu.SemaphoreType.DMA,
        ],
    )(_body)
    return k(idx_flat, tab_pad)


def kernel(numerical, cat_indices, tables):
    offs = jnp.arange(_N_CAT, dtype=jnp.int32) * jnp.int32(_VOCAB)
    idx_flat = (cat_indices + offs[None, :]).reshape(_FLAT)
    tab_pad = jnp.pad(
        tables.reshape(_N_CAT * _VOCAB, _DIM), ((0, 0), (0, _PDIM - _DIM))
    )
    rows = _run(idx_flat, tab_pad)
    emb = rows[:, :_DIM].reshape(_B, _N_CAT * _DIM)
    return jnp.concatenate([numerical, emb], axis=-1)
